# Initial kernel scaffold; baseline (speedup 1.0000x reference)
import jax
import jax.numpy as jnp
from jax.experimental import pallas as pl


def kernel(node_feats, edge_feats, params, edge_index):
    raise NotImplementedError("write your pallas kernel here")


# speedup vs baseline: 1.1374x; 1.1374x over previous
import functools
from typing import Sequence

import jax
import jax.numpy as jnp
from jax import lax
from jax.experimental import pallas as pl
from jax.experimental.pallas import tpu as pltpu

_BN_EPS = 1e-5


def _pick_bm(m, target):
    for bm in (target, 2048, 1600, 1280, 1024, 1000, 800, 640, 512, 400, 320,
               256, 200, 160, 128, 80, 64, 40, 32, 16, 8):
        if bm <= m and m % bm == 0 and bm % 8 == 0:
            return bm
    return m


def _linear_call(arrs, terms, b, *, act, want_stats, bm_target=1280):
    m = arrs[0].shape[0]
    dout = terms[0][4].shape[1]
    bm = _pick_bm(m, bm_target)
    grid = (m // bm,)

    n_arr = len(arrs)
    has_aff = [t[2] is not None for t in terms]

    def body(*refs):
        arr_refs = refs[:n_arr]
        k = n_arr
        term_data = []
        for (ia, ib, s, t, _w), aff in zip(terms, has_aff):
            s_ref = t_ref = None
            if aff:
                s_ref, t_ref = refs[k], refs[k + 1]
                k += 2
            w_ref = refs[k]
            k += 1
            term_data.append((ia, ib, s_ref, t_ref, w_ref))
        b_ref = refs[k]
        k += 1
        out_ref = refs[k]
        st_ref = refs[k + 1] if want_stats else None

        acc = jnp.zeros((bm, dout), jnp.float32) + b_ref[...]
        for (ia, ib, s_ref, t_ref, w_ref) in term_data:
            x = arr_refs[ia][...]
            if ib is not None:
                x = x - arr_refs[ib][...]
            if s_ref is not None:
                x = x * s_ref[...] + t_ref[...]
            acc = acc + jnp.dot(x, w_ref[...],
                                preferred_element_type=jnp.float32)
        if act == 'relu':
            acc = jnp.maximum(acc, 0.0)
        elif act == 'sigmoid':
            acc = jax.nn.sigmoid(acc)
        out_ref[...] = acc
        if want_stats:
            s1 = jnp.sum(acc, axis=0, keepdims=True)
            s2 = jnp.sum(acc * acc, axis=0, keepdims=True)
            z = jnp.concatenate([s1, s2], axis=0)
            i = pl.program_id(0)

            @pl.when(i == 0)
            def _():
                st_ref[...] = z

            @pl.when(i > 0)
            def _():
                st_ref[...] += z

    in_specs = []
    inputs = []
    for a in arrs:
        inputs.append(a)
        in_specs.append(pl.BlockSpec((bm, a.shape[1]), lambda i: (i, 0)))
    for (ia, ib, s, t, w), aff in zip(terms, has_aff):
        din = w.shape[0]
        if aff:
            inputs += [s.reshape(1, din), t.reshape(1, din)]
            in_specs += [pl.BlockSpec((1, din), lambda i: (0, 0))] * 2
        inputs.append(w)
        in_specs.append(pl.BlockSpec((din, dout), lambda i: (0, 0)))
    inputs.append(b.reshape(1, dout))
    in_specs.append(pl.BlockSpec((1, dout), lambda i: (0, 0)))

    out_shape = [jax.ShapeDtypeStruct((m, dout), jnp.float32)]
    out_specs = [pl.BlockSpec((bm, dout), lambda i: (i, 0))]
    if want_stats:
        out_shape.append(jax.ShapeDtypeStruct((2, dout), jnp.float32))
        out_specs.append(pl.BlockSpec((2, dout), lambda i: (0, 0)))

    res = pl.pallas_call(
        body, grid=grid, in_specs=in_specs, out_specs=out_specs,
        out_shape=out_shape)(*inputs)
    return (res[0], res[1]) if want_stats else (res[0], None)


def _colstats_call(specs, *, bm_target=1280):
    m = specs[0][0].shape[0]
    bm = _pick_bm(m, bm_target)
    grid = (m // bm,)
    n_out = len(specs)

    flat = []
    layout = []
    for sp in specs:
        layout.append((len(flat), len(sp) == 2))
        flat.extend(sp)

    def body(*refs):
        in_refs = refs[:len(flat)]
        out_refs = refs[len(flat):]
        i = pl.program_id(0)
        for (start, has_b), o_ref in zip(layout, out_refs):
            x = in_refs[start][...]
            if has_b:
                x = x - in_refs[start + 1][...]
            s1 = jnp.sum(x, axis=0, keepdims=True)
            s2 = jnp.sum(x * x, axis=0, keepdims=True)
            z = jnp.concatenate([s1, s2], axis=0)

            @pl.when(i == 0)
            def _(o_ref=o_ref, z=z):
                o_ref[...] = z

            @pl.when(i > 0)
            def _(o_ref=o_ref, z=z):
                o_ref[...] += z

    in_specs = [pl.BlockSpec((bm, a.shape[1]), lambda i: (i, 0)) for a in flat]
    out_shape = [jax.ShapeDtypeStruct((2, sp[0].shape[1]), jnp.float32)
                 for sp in specs]
    out_specs = [pl.BlockSpec((2, sp[0].shape[1]), lambda i: (0, 0))
                 for sp in specs]
    res = pl.pallas_call(body, grid=grid, in_specs=in_specs,
                         out_specs=out_specs, out_shape=out_shape)(*flat)
    return list(res)


def _bn_affine(stats, m):
    mu = stats[0] / m
    var = stats[1] / m - mu * mu
    s = lax.rsqrt(var + _BN_EPS)
    return s, -mu * s


def _compose_affine(s_in, t_in, s_out, t_out):
    return s_in * s_out, t_in * s_out + t_out


def _affine_stats(stats, s, t, m):
    s1, s2 = stats[0], stats[1]
    return jnp.stack([s * s1 + m * t,
                      s * s * s2 + 2.0 * s * t * s1 + m * t * t])


def _gather_rows(table, idx):
    return table[idx]


def _segment_max_affine(msg, dst, n, s, t):
    agg = jax.ops.segment_max(msg, dst, num_segments=n)
    cnt = jax.ops.segment_sum(jnp.ones((msg.shape[0],), jnp.float32), dst,
                              num_segments=n)
    agg = jnp.where(jnp.isfinite(agg), agg, 0.0)
    return jnp.where((cnt > 0)[:, None], agg * s + t, 0.0)


def _mlp3_edge(arrs, terms_in, p, *, e_rows):
    w1, b1 = p['lin1']['W'], p['lin1']['b']
    terms = []
    off = 0
    for (ia, ib, s, t) in terms_in:
        din = arrs[ia].shape[1]
        terms.append((ia, ib, s, t, w1[off:off + din]))
        off += din
    y1, st1 = _linear_call(arrs, terms, b1, act='relu', want_stats=True)
    s1, t1 = _bn_affine(st1, e_rows)
    y2, st2 = _linear_call([y1], [(0, None, s1, t1, p['lin2']['W'])],
                           p['lin2']['b'], act='relu', want_stats=True)
    s2, t2 = _bn_affine(st2, e_rows)
    y3, st3 = _linear_call([y2], [(0, None, s2, t2, p['lin3']['W'])],
                           p['lin3']['b'], act='relu', want_stats=True)
    s3, t3 = _bn_affine(st3, e_rows)
    return y3, (s3, t3), st3


def kernel(node_feats, edge_feats, params, edge_index):
    src = edge_index[0]
    dst = edge_index[1]
    n = node_feats.shape[0]
    e = src.shape[0]
    ef32 = jnp.float32(e)

    xd0 = _gather_rows(node_feats, dst)
    xs0 = _gather_rows(node_feats, src)
    st_a, st_b = _colstats_call([(xd0,), (xs0, xd0)])
    s0a, t0a = _bn_affine(st_a, ef32)
    s0b, t0b = _bn_affine(st_b, ef32)
    y3, (s3, t3), _ = _mlp3_edge(
        [xd0, xs0], [(0, None, s0a, t0a), (1, 0, s0b, t0b)],
        params['nmm1'], e_rows=ef32)
    x1 = _segment_max_affine(y3, dst, n, s3, t3)

    xs1 = _gather_rows(x1, src)
    xd1 = _gather_rows(x1, dst)
    st_e0, st_s1, st_d1 = _colstats_call([(edge_feats,), (xs1,), (xd1,)])
    se0, te0 = _bn_affine(st_e0, ef32)
    ss1, ts1 = _bn_affine(st_s1, ef32)
    sd1, td1 = _bn_affine(st_d1, ef32)
    e1, (es3, et3), est3 = _mlp3_edge(
        [edge_feats, xs1, xd1],
        [(0, None, se0, te0), (1, None, ss1, ts1), (2, None, sd1, td1)],
        params['emm1'], e_rows=ef32)

    z3, (zs3, zt3), _ = _mlp3_edge(
        [xd1, xs1], [(0, None, None, None), (1, 0, None, None)],
        params['nmm2'], e_rows=ef32)
    x2 = _segment_max_affine(z3, dst, n, zs3, zt3)

    xs2 = _gather_rows(x2, src)
    xd2 = _gather_rows(x2, dst)
    st_s2, st_d2 = _colstats_call([(xs2,), (xd2,)])
    st_e1p = _affine_stats(est3, es3, et3, ef32)
    se1, te1 = _bn_affine(st_e1p, ef32)
    se1c, te1c = _compose_affine(es3, et3, se1, te1)
    ss2, ts2 = _bn_affine(st_s2, ef32)
    sd2, td2 = _bn_affine(st_d2, ef32)
    e2, (fs3, ft3), _ = _mlp3_edge(
        [e1, xs2, xd2],
        [(0, None, se1c, te1c), (1, None, ss2, ts2), (2, None, sd2, td2)],
        params['emm2'], e_rows=ef32)

    ph = params['nhead']
    h1, _ = _linear_call([x2], [(0, None, None, None, ph['l1']['W'])],
                         ph['l1']['b'], act='relu', want_stats=False,
                         bm_target=1000)
    h2, _ = _linear_call([h1], [(0, None, None, None, ph['l2']['W'])],
                         ph['l2']['b'], act='relu', want_stats=False,
                         bm_target=1000)
    w34 = ph['l3']['W'] @ ph['l4']['W']
    b34 = ph['l3']['b'] @ ph['l4']['W'] + ph['l4']['b']
    n_out, _ = _linear_call([h2], [(0, None, None, None, w34)], b34,
                            act='sigmoid', want_stats=False, bm_target=1000)

    pe = params['ehead']
    w12 = pe['l1']['W'] @ pe['l2']['W']
    b12 = pe['l1']['b'] @ pe['l2']['W'] + pe['l2']['b']
    w12f = fs3.reshape(-1, 1) * w12
    b12f = ft3 @ w12 + b12
    g1, _ = _linear_call([e2], [(0, None, None, None, w12f)], b12f,
                         act='relu', want_stats=False)
    g2, _ = _linear_call([g1], [(0, None, None, None, pe['l3']['W'])],
                         pe['l3']['b'], act='relu', want_stats=False)
    w45 = pe['l4']['W'] @ pe['l5']['W']
    b45 = pe['l4']['b'] @ pe['l5']['W'] + pe['l5']['b']
    e_out, _ = _linear_call([g2], [(0, None, None, None, w45)], b45,
                            act='sigmoid', want_stats=False)

    return (n_out, e_out)


# speedup vs baseline: 1.3742x; 1.2082x over previous
import functools
from typing import Sequence

import jax
import jax.numpy as jnp
from jax import lax
from jax.experimental import pallas as pl
from jax.experimental.pallas import tpu as pltpu
from jax.experimental.pallas import tpu_sc as plsc

_BN_EPS = 1e-5
_NW = 32


def _pick_bm(m, target):
    for bm in (target, 2048, 1600, 1280, 1024, 1000, 800, 640, 512, 400, 320,
               256, 200, 160, 128, 80, 64, 40, 32, 16, 8):
        if bm <= m and m % bm == 0 and bm % 8 == 0:
            return bm
    return m


def _linear_call(arrs, terms, b, *, act, want_stats, bm_target=1280,
                 nsplit=1):
    m = arrs[0].shape[0]
    dout = terms[0][4].shape[1]
    dpart = dout // nsplit
    bm = _pick_bm(m, bm_target)
    grid = (m // bm,)

    n_arr = len(arrs)
    has_aff = [t[2] is not None for t in terms]

    def body(*refs):
        arr_refs = refs[:n_arr]
        k = n_arr
        term_data = []
        for (ia, ib, s, t, _w), aff in zip(terms, has_aff):
            s_ref = t_ref = None
            if aff:
                s_ref, t_ref = refs[k], refs[k + 1]
                k += 2
            w_ref = refs[k]
            k += 1
            term_data.append((ia, ib, s_ref, t_ref, w_ref))
        b_ref = refs[k]
        k += 1
        out_refs = refs[k:k + nsplit]
        st_ref = refs[k + nsplit] if want_stats else None

        acc = jnp.zeros((bm, dout), jnp.float32) + b_ref[...]
        for (ia, ib, s_ref, t_ref, w_ref) in term_data:
            x = arr_refs[ia][...]
            if ib is not None:
                x = x - arr_refs[ib][...]
            if s_ref is not None:
                x = x * s_ref[...] + t_ref[...]
            acc = acc + jnp.dot(x, w_ref[...],
                                preferred_element_type=jnp.float32)
        if act == 'relu':
            acc = jnp.maximum(acc, 0.0)
        elif act == 'sigmoid':
            acc = jax.nn.sigmoid(acc)
        for p_i, o_ref in enumerate(out_refs):
            o_ref[...] = acc[:, p_i * dpart:(p_i + 1) * dpart]
        if want_stats:
            s1 = jnp.sum(acc, axis=0, keepdims=True)
            s2 = jnp.sum(acc * acc, axis=0, keepdims=True)
            z = jnp.concatenate([s1, s2], axis=0)
            i = pl.program_id(0)

            @pl.when(i == 0)
            def _():
                st_ref[...] = z

            @pl.when(i > 0)
            def _():
                st_ref[...] += z

    in_specs = []
    inputs = []
    for a in arrs:
        inputs.append(a)
        in_specs.append(pl.BlockSpec((bm, a.shape[1]), lambda i: (i, 0)))
    for (ia, ib, s, t, w), aff in zip(terms, has_aff):
        din = w.shape[0]
        if aff:
            inputs += [s.reshape(1, din), t.reshape(1, din)]
            in_specs += [pl.BlockSpec((1, din), lambda i: (0, 0))] * 2
        inputs.append(w)
        in_specs.append(pl.BlockSpec((din, dout), lambda i: (0, 0)))
    inputs.append(b.reshape(1, dout))
    in_specs.append(pl.BlockSpec((1, dout), lambda i: (0, 0)))

    out_shape = [jax.ShapeDtypeStruct((m, dpart), jnp.float32)] * nsplit
    out_specs = [pl.BlockSpec((bm, dpart), lambda i: (i, 0))] * nsplit
    if want_stats:
        out_shape.append(jax.ShapeDtypeStruct((2, dout), jnp.float32))
        out_specs.append(pl.BlockSpec((2, dout), lambda i: (0, 0)))

    res = pl.pallas_call(
        body, grid=grid, in_specs=in_specs, out_specs=out_specs,
        out_shape=out_shape)(*inputs)
    outs = res[0] if nsplit == 1 else list(res[:nsplit])
    return (outs, res[nsplit]) if want_stats else (outs, None)


def _colstats_call(specs, *, bm_target=1280):
    m = specs[0][0].shape[0]
    bm = _pick_bm(m, bm_target)
    grid = (m // bm,)
    n_out = len(specs)

    flat = []
    layout = []
    for sp in specs:
        layout.append((len(flat), len(sp) == 2))
        flat.extend(sp)

    def body(*refs):
        in_refs = refs[:len(flat)]
        out_refs = refs[len(flat):]
        i = pl.program_id(0)
        for (start, has_b), o_ref in zip(layout, out_refs):
            x = in_refs[start][...]
            if has_b:
                x = x - in_refs[start + 1][...]
            s1 = jnp.sum(x, axis=0, keepdims=True)
            s2 = jnp.sum(x * x, axis=0, keepdims=True)
            z = jnp.concatenate([s1, s2], axis=0)

            @pl.when(i == 0)
            def _(o_ref=o_ref, z=z):
                o_ref[...] = z

            @pl.when(i > 0)
            def _(o_ref=o_ref, z=z):
                o_ref[...] += z

    in_specs = [pl.BlockSpec((bm, a.shape[1]), lambda i: (i, 0)) for a in flat]
    out_shape = [jax.ShapeDtypeStruct((2, sp[0].shape[1]), jnp.float32)
                 for sp in specs]
    out_specs = [pl.BlockSpec((2, sp[0].shape[1]), lambda i: (0, 0))
                 for sp in specs]
    res = pl.pallas_call(body, grid=grid, in_specs=in_specs,
                         out_specs=out_specs, out_shape=out_shape)(*flat)
    return list(res)


def _bn_affine(stats, m):
    mu = stats[0] / m
    var = stats[1] / m - mu * mu
    s = lax.rsqrt(var + _BN_EPS)
    return s, -mu * s


def _compose_affine(s_in, t_in, s_out, t_out):
    return s_in * s_out, t_in * s_out + t_out


def _affine_stats(stats, s, t, m):
    s1, s2 = stats[0], stats[1]
    return jnp.stack([s * s1 + m * t,
                      s * s * s2 + 2.0 * s * t * s1 + m * t * t])


def _sc_gather2(table, idx0, idx1):
    e = idx0.shape[0]
    d = table.shape[1]
    per_w = e // _NW
    r = next(c for c in (200, 80, 40, 8) if per_w % c == 0)
    n_chunks = per_w // r
    mesh = plsc.VectorSubcoreMesh(core_axis_name="c", subcore_axis_name="s")

    @functools.partial(
        pl.kernel,
        out_type=[jax.ShapeDtypeStruct((e, d), jnp.float32)] * 2,
        mesh=mesh,
        scratch_types=[
            pltpu.VMEM((per_w,), jnp.int32),
            pltpu.VMEM((r, d), jnp.float32),
            pltpu.SemaphoreType.DMA,
        ],
    )
    def k(table_hbm, i0_hbm, i1_hbm, o0_hbm, o1_hbm, idx_v, rows_v, sem):
        wid = lax.axis_index("s") * 2 + lax.axis_index("c")
        base = wid * per_w
        for i_hbm, o_hbm in ((i0_hbm, o0_hbm), (i1_hbm, o1_hbm)):
            pltpu.sync_copy(i_hbm.at[pl.ds(base, per_w)], idx_v)

            def body(c, _, o_hbm=o_hbm):
                pltpu.async_copy(
                    table_hbm.at[idx_v.at[pl.ds(c * r, r)]], rows_v,
                    sem).wait()
                pltpu.sync_copy(rows_v, o_hbm.at[pl.ds(base + c * r, r)])
                return _

            lax.fori_loop(0, n_chunks, body, 0)

    return k(table, idx0, idx1)


def _segment_max_affine(msg, dst, s, t, n_nodes):
    agg = jax.ops.segment_max(msg, dst, num_segments=n_nodes)
    return jnp.where(jnp.isfinite(agg), agg * s + t, 0.0)


def _mlp3_edge(arrs, terms_in, w1_list, p, *, e_rows, nsplit_out=1):
    b1 = p['lin1']['b']
    terms = [(ia, ib, s, t, w)
             for (ia, ib, s, t), w in zip(terms_in, w1_list)]
    y1, st1 = _linear_call(arrs, terms, b1, act='relu', want_stats=True)
    s1, t1 = _bn_affine(st1, e_rows)
    y2, st2 = _linear_call([y1], [(0, None, s1, t1, p['lin2']['W'])],
                           p['lin2']['b'], act='relu', want_stats=True)
    s2, t2 = _bn_affine(st2, e_rows)
    y3, st3 = _linear_call([y2], [(0, None, s2, t2, p['lin3']['W'])],
                           p['lin3']['b'], act='relu', want_stats=True,
                           nsplit=nsplit_out)
    s3, t3 = _bn_affine(st3, e_rows)
    return y3, (s3, t3), st3


def kernel(node_feats, edge_feats, params, edge_index):
    src = edge_index[0]
    dst = edge_index[1]
    n = node_feats.shape[0]
    e = src.shape[0]
    ef32 = jnp.float32(e)

    d0 = node_feats.shape[1]
    pad0 = (-d0) % 128
    nf = jnp.pad(node_feats, ((0, 0), (0, pad0)))
    w1n = params['nmm1']['lin1']['W']
    zpad = jnp.zeros((pad0, w1n.shape[1]), jnp.float32)
    w1n_parts = [jnp.concatenate([w1n[:d0], zpad]),
                 jnp.concatenate([w1n[d0:], zpad])]
    xd0, xs0 = _sc_gather2(nf, dst, src)
    st_a, st_b = _colstats_call([(xd0,), (xs0, xd0)])
    s0a, t0a = _bn_affine(st_a, ef32)
    s0b, t0b = _bn_affine(st_b, ef32)
    y3, (s3, t3), _ = _mlp3_edge(
        [xd0, xs0], [(0, None, s0a, t0a), (1, 0, s0b, t0b)], w1n_parts,
        params['nmm1'], e_rows=ef32)
    x1 = _segment_max_affine(y3, dst, s3, t3, n)

    xs1, xd1 = _sc_gather2(x1, src, dst)
    st_e0, st_s1, st_d1 = _colstats_call([(edge_feats,), (xs1,), (xd1,)])
    se0, te0 = _bn_affine(st_e0, ef32)
    ss1, ts1 = _bn_affine(st_s1, ef32)
    sd1, td1 = _bn_affine(st_d1, ef32)
    w1e = params['emm1']['lin1']['W']
    de0, d1 = edge_feats.shape[1], xs1.shape[1]
    e1, (es3, et3), est3 = _mlp3_edge(
        [edge_feats, xs1, xd1],
        [(0, None, se0, te0), (1, None, ss1, ts1), (2, None, sd1, td1)],
        [w1e[:de0], w1e[de0:de0 + d1], w1e[de0 + d1:]],
        params['emm1'], e_rows=ef32)

    w1n2 = params['nmm2']['lin1']['W']
    z3, (zs3, zt3), _ = _mlp3_edge(
        [xd1, xs1], [(0, None, None, None), (1, 0, None, None)],
        [w1n2[:d1], w1n2[d1:]], params['nmm2'], e_rows=ef32)
    x2 = _segment_max_affine(z3, dst, zs3, zt3, n)

    xs2, xd2 = _sc_gather2(x2, src, dst)
    st_s2, st_d2 = _colstats_call([(xs2,), (xd2,)])
    st_e1p = _affine_stats(est3, es3, et3, ef32)
    se1, te1 = _bn_affine(st_e1p, ef32)
    se1c, te1c = _compose_affine(es3, et3, se1, te1)
    ss2, ts2 = _bn_affine(st_s2, ef32)
    sd2, td2 = _bn_affine(st_d2, ef32)
    w1e2 = params['emm2']['lin1']['W']
    de1, d2 = e1.shape[1], xs2.shape[1]
    e2, (fs3, ft3), _ = _mlp3_edge(
        [e1, xs2, xd2],
        [(0, None, se1c, te1c), (1, None, ss2, ts2), (2, None, sd2, td2)],
        [w1e2[:de1], w1e2[de1:de1 + d2], w1e2[de1 + d2:]],
        params['emm2'], e_rows=ef32)

    ph = params['nhead']
    h1, _ = _linear_call([x2], [(0, None, None, None, ph['l1']['W'])],
                         ph['l1']['b'], act='relu', want_stats=False,
                         bm_target=1000)
    h2, _ = _linear_call([h1], [(0, None, None, None, ph['l2']['W'])],
                         ph['l2']['b'], act='relu', want_stats=False,
                         bm_target=1000)
    w34 = ph['l3']['W'] @ ph['l4']['W']
    b34 = ph['l3']['b'] @ ph['l4']['W'] + ph['l4']['b']
    n_out, _ = _linear_call([h2], [(0, None, None, None, w34)], b34,
                            act='sigmoid', want_stats=False, bm_target=1000)

    pe = params['ehead']
    w12 = pe['l1']['W'] @ pe['l2']['W']
    b12 = pe['l1']['b'] @ pe['l2']['W'] + pe['l2']['b']
    w12f = fs3.reshape(-1, 1) * w12
    b12f = ft3 @ w12 + b12
    g1, _ = _linear_call([e2], [(0, None, None, None, w12f)], b12f,
                         act='relu', want_stats=False)
    g2, _ = _linear_call([g1], [(0, None, None, None, pe['l3']['W'])],
                         pe['l3']['b'], act='relu', want_stats=False)
    w45 = pe['l4']['W'] @ pe['l5']['W']
    b45 = pe['l4']['b'] @ pe['l5']['W'] + pe['l5']['b']
    e_out, _ = _linear_call([g2], [(0, None, None, None, w45)], b45,
                            act='sigmoid', want_stats=False)

    return (n_out, e_out)
